# baseline (speedup 1.0000x reference)
import functools

import jax
import jax.numpy as jnp
from jax import lax
from jax.experimental import pallas as pl
from jax.experimental.pallas import tpu as pltpu
from jax.experimental.pallas import tpu_sc as plsc

NUM = 1000000
K = 16
BATCH = 16384

NUM_CORES = 2
NUM_SUBCORES = 16
NUM_WORKERS = NUM_CORES * NUM_SUBCORES
BPW = BATCH // NUM_WORKERS

_mesh = plsc.VectorSubcoreMesh(core_axis_name="c", subcore_axis_name="s")


@functools.partial(
    pl.kernel,
    mesh=_mesh,
    out_type=jax.ShapeDtypeStruct((BATCH, K), jnp.float32),
    scratch_types=[
        pltpu.VMEM((BPW,), jnp.int32),
        pltpu.VMEM((BPW, K), jnp.float32),
        pltpu.SemaphoreType.DMA,
    ],
    compiler_params=pltpu.CompilerParams(use_tc_tiling_on_sc=False),
)
def _lookup_softmax(uid_hbm, table_hbm, out_hbm, idx_v, rows_v, sem):
    wid = lax.axis_index("s") * NUM_CORES + lax.axis_index("c")
    base = wid * BPW
    pltpu.sync_copy(uid_hbm.at[pl.ds(base, BPW)], idx_v)
    pltpu.async_copy(table_hbm.at[idx_v], rows_v, sem).wait()

    lanes = lax.iota(jnp.int32, K)

    def _perm(x, idx):
        return lax.gather(
            x,
            idx[:, None],
            dimension_numbers=lax.GatherDimensionNumbers(
                offset_dims=(), collapsed_slice_dims=(0,), start_index_map=(0,)
            ),
            slice_sizes=(1,),
            mode=lax.GatherScatterMode.PROMISE_IN_BOUNDS,
        )

    def body(i, carry):
        row = rows_v[i, :]
        m = row
        for k in range(4):
            m = jnp.maximum(m, _perm(m, lanes ^ (1 << k)))
        e = jnp.exp(row - m)
        s = e
        for k in range(4):
            s = s + _perm(s, lanes ^ (1 << k))
        rows_v[i, :] = e / s
        return carry

    lax.fori_loop(0, BPW, body, 0)
    pltpu.sync_copy(rows_v, out_hbm.at[pl.ds(base, BPW)])


def kernel(uid, matrix):
    return _lookup_softmax(uid.astype(jnp.int32), matrix)


# speedup vs baseline: 1.0224x; 1.0224x over previous
import functools

import jax
import jax.numpy as jnp
from jax import lax
from jax.experimental import pallas as pl
from jax.experimental.pallas import tpu as pltpu
from jax.experimental.pallas import tpu_sc as plsc

NUM = 1000000
K = 16
BATCH = 16384

NUM_CORES = 2
NUM_SUBCORES = 16
NUM_WORKERS = NUM_CORES * NUM_SUBCORES
BPW = BATCH // NUM_WORKERS
SLABS_PER_W = BPW // 8

_mesh = plsc.VectorSubcoreMesh(core_axis_name="c", subcore_axis_name="s")


@functools.partial(
    pl.kernel,
    mesh=_mesh,
    out_type=jax.ShapeDtypeStruct((BATCH // 8, 128), jnp.float32),
    scratch_types=[
        pltpu.VMEM((BPW,), jnp.int32),
        pltpu.VMEM((BPW,), jnp.int32),
        pltpu.VMEM((BPW, 128), jnp.float32),
        pltpu.VMEM((SLABS_PER_W, 128), jnp.float32),
        pltpu.SemaphoreType.DMA,
    ],
    compiler_params=pltpu.CompilerParams(needs_layout_passes=False),
)
def _lookup_softmax(uid_hbm, table_hbm, out_hbm, idx_v, slabidx_v, slab_v,
                    out_v, sem):
    wid = lax.axis_index("s") * NUM_CORES + lax.axis_index("c")
    base = wid * BPW
    pltpu.sync_copy(uid_hbm.at[pl.ds(base, BPW)], idx_v)

    lanes = lax.iota(jnp.int32, 16)

    def idx_body(c, carry):
        v = idx_v[pl.ds(c * 16, 16)]
        slabidx_v[pl.ds(c * 16, 16)] = v >> 3
        return carry

    lax.fori_loop(0, BPW // 16, idx_body, 0)

    pltpu.async_copy(table_hbm.at[slabidx_v], slab_v, sem).wait()

    def body(b, carry):
        uids = idx_v[pl.ds(b * 16, 16)]
        subcol = (uids & 7) << 4
        srow = b * 16 + lanes
        cols = [plsc.load_gather(slab_v, [srow, subcol + j]) for j in range(K)]
        m = cols[0]
        for j in range(1, K):
            m = jnp.maximum(m, cols[j])
        es = [jnp.exp(c - m) for c in cols]
        s = es[0]
        for j in range(1, K):
            s = s + es[j]
        inv = 1.0 / s
        orow = (b * 2) + (lanes >> 3)
        ocol = (lanes & 7) << 4
        for j in range(K):
            plsc.store_scatter(out_v, [orow, ocol + j], es[j] * inv)
        return carry

    lax.fori_loop(0, BPW // 16, body, 0)
    pltpu.sync_copy(out_v, out_hbm.at[pl.ds(wid * SLABS_PER_W, SLABS_PER_W)])


def kernel(uid, matrix):
    packed = _lookup_softmax(uid.astype(jnp.int32),
                             matrix.reshape(NUM // 8, 128))
    return packed.reshape(BATCH, K)


# speedup vs baseline: 1.0552x; 1.0321x over previous
import functools

import jax
import jax.numpy as jnp
from jax import lax
from jax.experimental import pallas as pl
from jax.experimental.pallas import tpu as pltpu
from jax.experimental.pallas import tpu_sc as plsc

NUM = 1000000
K = 16
BATCH = 16384

NUM_CORES = 2
NUM_SUBCORES = 16
NUM_WORKERS = NUM_CORES * NUM_SUBCORES
BPW = BATCH // NUM_WORKERS

_mesh = plsc.VectorSubcoreMesh(core_axis_name="c", subcore_axis_name="s")


@functools.partial(
    pl.kernel,
    mesh=_mesh,
    out_type=jax.ShapeDtypeStruct((K, BATCH), jnp.float32),
    scratch_types=[
        pltpu.VMEM((BPW,), jnp.int32),
        pltpu.VMEM((BPW,), jnp.int32),
        pltpu.VMEM((BPW, 128), jnp.float32),
        pltpu.VMEM((K, BPW), jnp.float32),
        pltpu.SemaphoreType.DMA,
    ],
    compiler_params=pltpu.CompilerParams(needs_layout_passes=False),
)
def _lookup_softmax(uid_hbm, table_hbm, out_t_hbm, idx_v, slabidx_v, slab_v,
                    res_v, sem):
    wid = lax.axis_index("s") * NUM_CORES + lax.axis_index("c")
    base = wid * BPW
    pltpu.sync_copy(uid_hbm.at[pl.ds(base, BPW)], idx_v)

    lanes = lax.iota(jnp.int32, 16)

    def idx_body(c, carry):
        v = idx_v[pl.ds(c * 16, 16)]
        slabidx_v[pl.ds(c * 16, 16)] = v >> 3
        return carry

    lax.fori_loop(0, BPW // 16, idx_body, 0)

    pltpu.async_copy(table_hbm.at[slabidx_v], slab_v, sem).wait()

    def body(b, carry):
        uvec = idx_v[pl.ds(b * 16, 16)]
        subcol = (uvec & 7) << 4
        srow = b * 16 + lanes
        cols = [plsc.load_gather(slab_v, [srow, subcol + j]) for j in range(K)]
        m = cols[0]
        for j in range(1, K):
            m = jnp.maximum(m, cols[j])
        es = [jnp.exp(c - m) for c in cols]
        s = es[0]
        for j in range(1, K):
            s = s + es[j]
        inv = 1.0 / s
        ocol = b * 16 + lanes
        for j in range(K):
            plsc.store_scatter(
                res_v, [jnp.full((16,), j, jnp.int32), ocol], es[j] * inv)
        return carry

    lax.fori_loop(0, BPW // 16, body, 0)
    for h in range(2):
        pltpu.sync_copy(res_v.at[pl.ds(h * 8, 8), :],
                        out_t_hbm.at[pl.ds(h * 8, 8), pl.ds(base, BPW)])


def kernel(uid, matrix):
    out_t = _lookup_softmax(uid.astype(jnp.int32),
                            matrix.reshape(NUM // 8, 128))
    return out_t.T


# speedup vs baseline: 3.6676x; 3.4757x over previous
import functools

import jax
import jax.numpy as jnp
from jax import lax
from jax.experimental import pallas as pl
from jax.experimental.pallas import tpu as pltpu
from jax.experimental.pallas import tpu_sc as plsc

NUM = 1000000
K = 16
BATCH = 16384

NUM_CORES = 2
NUM_SUBCORES = 16
NUM_WORKERS = NUM_CORES * NUM_SUBCORES
BPW = BATCH // NUM_WORKERS
CHUNK = 128
NCT = (NUM + 127) // 128
NTILES = 2 * NCT
NROUND = (NTILES + NUM_WORKERS - 1) // NUM_WORKERS
GRP = 8
UNITROWS = NTILES * 64

_mesh = plsc.VectorSubcoreMesh(core_axis_name="c", subcore_axis_name="s")


@functools.partial(
    pl.kernel,
    mesh=_mesh,
    out_type=jax.ShapeDtypeStruct((NTILES, 8, 128), jnp.float32),
    scratch_types=[
        pltpu.VMEM((GRP, 8, 128), jnp.float32),
        pltpu.SemaphoreType.DMA,
        pltpu.SemaphoreType.DMA,
    ],
    compiler_params=pltpu.CompilerParams(needs_layout_passes=False),
)
def _relayout(table_t_hbm, lin_hbm, buf_v, sem_in, sem_out):
    wid = lax.axis_index("s") * NUM_CORES + lax.axis_index("c")

    def group_body(g, carry):
        ts = []
        for i in range(GRP):
            t = wid + (g * GRP + i) * NUM_WORKERS
            h = jnp.where(t >= NCT, 1, 0)
            j = t - h * NCT
            ts.append((t, h, j))
        ins = []
        for i, (t, h, j) in enumerate(ts):
            src = table_t_hbm.at[
                pl.ds(pl.multiple_of(h * 8, 8), 8),
                pl.ds(pl.multiple_of(j * 128, 128), 128),
            ]
            cp = pltpu.make_async_copy(src, buf_v.at[i], sem_in)

            @pl.when(t < NTILES)
            def _():
                cp.start()

            ins.append((cp, t))
        for cp, t in ins:
            @pl.when(t < NTILES)
            def _():
                cp.wait()

        outs = []
        for i, (t, h, j) in enumerate(ts):
            cp = pltpu.make_async_copy(buf_v.at[i], lin_hbm.at[t], sem_out)

            @pl.when(t < NTILES)
            def _():
                cp.start()

            outs.append((cp, t))
        for cp, t in outs:
            @pl.when(t < NTILES)
            def _():
                cp.wait()

        return carry

    lax.fori_loop(0, (NROUND + GRP - 1) // GRP, group_body, 0)


@functools.partial(
    pl.kernel,
    mesh=_mesh,
    out_type=jax.ShapeDtypeStruct((K, BATCH), jnp.float32),
    scratch_types=[
        pltpu.VMEM((BPW,), jnp.int32),
        pltpu.VMEM((K * CHUNK,), jnp.int32),
        pltpu.VMEM((K * CHUNK, 16), jnp.float32),
        pltpu.VMEM((K, BPW), jnp.float32),
        pltpu.SemaphoreType.DMA,
    ],
    compiler_params=pltpu.CompilerParams(
        needs_layout_passes=False, use_tc_tiling_on_sc=False),
)
def _lookup_softmax(uid_hbm, tbl_hbm, out_t_hbm, idx_v, gidx_v, gbuf_v,
                    res_v, sem):
    wid = lax.axis_index("s") * NUM_CORES + lax.axis_index("c")
    base = wid * BPW
    pltpu.sync_copy(uid_hbm.at[pl.ds(base, BPW)], idx_v)

    lanes = lax.iota(jnp.int32, 16)

    def chunk_body(c, carry):
        def build(bb, carry2):
            uvec = idx_v[pl.ds(c * CHUNK + bb * 16, 16)]
            tvec = ((uvec >> 7) << 6) + ((uvec >> 4) & 7)
            for f in range(K):
                off = (f >> 3) * (NCT * 64) + ((f & 7) << 3)
                gidx_v[pl.ds(f * CHUNK + bb * 16, 16)] = tvec + off
            return carry2

        lax.fori_loop(0, CHUNK // 16, build, 0)
        pltpu.async_copy(tbl_hbm.at[gidx_v], gbuf_v, sem).wait()

        def soft(bb, carry2):
            uvec = idx_v[pl.ds(c * CHUNK + bb * 16, 16)]
            umod = uvec & 15
            rows = bb * 16 + lanes
            cols = [
                plsc.load_gather(gbuf_v, [rows + f * CHUNK, umod])
                for f in range(K)
            ]
            m = cols[0]
            for f in range(1, K):
                m = jnp.maximum(m, cols[f])
            es = [jnp.exp(x - m) for x in cols]
            s = es[0]
            for f in range(1, K):
                s = s + es[f]
            inv = 1.0 / s
            ocol = c * CHUNK + bb * 16 + lanes
            for f in range(K):
                plsc.store_scatter(
                    res_v, [jnp.full((16,), f, jnp.int32), ocol], es[f] * inv)
            return carry2

        lax.fori_loop(0, CHUNK // 16, soft, 0)
        return carry

    lax.fori_loop(0, BPW // CHUNK, chunk_body, 0)

    for h in range(2):
        pltpu.sync_copy(res_v.at[pl.ds(h * 8, 8), :],
                        out_t_hbm.at[pl.ds(h * 8, 8), pl.ds(base, BPW)])


def kernel(uid, matrix):
    lin = _relayout(matrix.T)
    out_t = _lookup_softmax(uid.astype(jnp.int32), lin.reshape(UNITROWS, K))
    return out_t.T
